# speedup vs baseline: 1.8935x; 1.8935x over previous
import jax
import jax.numpy as jnp
from jax import lax
from jax.experimental import pallas as pl
from jax.experimental.pallas import tpu as pltpu
from jax.experimental.pallas import tpu_sc as plsc

_N_NODES = 10000
_N_EDGES = 320000
_D = 128
_NW = 32
_EPW = _N_EDGES // _NW
_C = 80
_NCHUNK = _EPW // _C
_L = 16


def _sc_body(px_hbm, ind_i_hbm, ind_j_hbm, wi_hbm, wj_hbm, out_hbm,
             idx_i, idx_j, buf_i, buf_j, wi_v, wj_v, sem_i, sem_j):
    cid = lax.axis_index("c")
    sid = lax.axis_index("s")
    wid = sid * 2 + cid
    base = wid * _EPW

    pltpu.sync_copy(wi_hbm, wi_v)
    pltpu.sync_copy(wj_hbm, wj_v)

    def chunk_body(k, carry):
        off = base + k * _C
        pltpu.sync_copy(ind_i_hbm.at[pl.ds(off, _C)], idx_i)
        pltpu.sync_copy(ind_j_hbm.at[pl.ds(off, _C)], idx_j)
        cp_i = pltpu.async_copy(px_hbm.at[idx_i], buf_i, sem_i)
        cp_j = pltpu.async_copy(px_hbm.at[idx_j], buf_j, sem_j)
        cp_i.wait()
        cp_j.wait()

        def row_body(e, carry2):
            for d in range(_D // _L):
                sl = pl.ds(d * _L, _L)
                a = buf_i[e, sl]
                b = buf_j[e, sl]
                buf_i[e, sl] = wi_v[sl] * a + wj_v[sl] * b
            return carry2

        lax.fori_loop(0, _C, row_body, 0)
        pltpu.sync_copy(buf_i, out_hbm.at[pl.ds(off, _C)])
        return carry

    lax.fori_loop(0, _NCHUNK, chunk_body, 0)


_pix_sc = pl.kernel(
    _sc_body,
    out_type=jax.ShapeDtypeStruct((_N_EDGES, _D), jnp.float32),
    mesh=plsc.VectorSubcoreMesh(core_axis_name="c", subcore_axis_name="s"),
    scratch_types=[
        pltpu.VMEM((_C,), jnp.int32),
        pltpu.VMEM((_C,), jnp.int32),
        pltpu.VMEM((_C, _D), jnp.float32),
        pltpu.VMEM((_C, _D), jnp.float32),
        pltpu.VMEM((_D,), jnp.float32),
        pltpu.VMEM((_D,), jnp.float32),
        pltpu.SemaphoreType.DMA,
        pltpu.SemaphoreType.DMA,
    ],
)


@jax.jit
def kernel(px, ind_2, wi, wj):
    ind_i = ind_2[:, 0]
    ind_j = ind_2[:, 1]
    return _pix_sc(px, ind_i, ind_j, wi, wj)


# speedup vs baseline: 6.0516x; 3.1961x over previous
import jax
import jax.numpy as jnp
from jax import lax
from jax.experimental import pallas as pl
from jax.experimental.pallas import tpu as pltpu
from jax.experimental.pallas import tpu_sc as plsc

_N_NODES = 10000
_N_EDGES = 320000
_D = 128
_NW = 32
_EPW = _N_EDGES // _NW
_C = 80
_NCHUNK = _EPW // _C
_NPAIR = (_NCHUNK - 1) // 2
_L = 16


def _sc_body(px_hbm, ind_i_hbm, ind_j_hbm, wi_hbm, wj_hbm, out_hbm,
             idx_i0, idx_j0, idx_i1, idx_j1,
             bi0, bj0, bi1, bj1, bo0, bo1,
             wi_v, wj_v,
             gsem0, gsem1, osem0, osem1):
    wid = lax.axis_index("s") * 2 + lax.axis_index("c")
    base = wid * _EPW

    pltpu.sync_copy(wi_hbm, wi_v)
    pltpu.sync_copy(wj_hbm, wj_v)
    wiv = [wi_v[pl.ds(d * _L, _L)] for d in range(_D // _L)]
    wjv = [wj_v[pl.ds(d * _L, _L)] for d in range(_D // _L)]

    idx = ((idx_i0, idx_j0), (idx_i1, idx_j1))
    bufs = ((bi0, bj0), (bi1, bj1))
    outs = (bo0, bo1)
    gsem = (gsem0, gsem1)
    osem = (osem0, osem1)

    def fire_gather(k, s):
        off = base + k * _C
        ii, jj = idx[s]
        pltpu.sync_copy(ind_i_hbm.at[pl.ds(off, _C)], ii)
        pltpu.sync_copy(ind_j_hbm.at[pl.ds(off, _C)], jj)
        pltpu.async_copy(px_hbm.at[ii], bufs[s][0], gsem[s])
        pltpu.async_copy(px_hbm.at[jj], bufs[s][1], gsem[s])

    def wait_gather(s):
        pltpu.make_async_copy(px_hbm.at[pl.ds(0, _C)], bufs[s][0], gsem[s]).wait()
        pltpu.make_async_copy(px_hbm.at[pl.ds(0, _C)], bufs[s][1], gsem[s]).wait()

    def compute(s):
        bi, bj = bufs[s]
        bo = outs[s]

        @plsc.parallel_loop(0, _C, unroll=4)
        def _(e):
            for d in range(_D // _L):
                sl = pl.ds(d * _L, _L)
                bo[e, sl] = wiv[d] * bi[e, sl] + wjv[d] * bj[e, sl]

    def fire_out(k, s):
        off = base + k * _C
        pltpu.async_copy(outs[s], out_hbm.at[pl.ds(off, _C)], osem[s])

    def wait_out(s):
        pltpu.make_async_copy(outs[s], out_hbm.at[pl.ds(0, _C)], osem[s]).wait()

    fire_gather(0, 0)

    def pair_body(g, carry):
        fire_gather(2 * g + 1, 1)
        wait_gather(0)

        @pl.when(g > 0)
        def _():
            wait_out(0)

        compute(0)
        fire_out(2 * g, 0)
        fire_gather(2 * g + 2, 0)
        wait_gather(1)

        @pl.when(g > 0)
        def _():
            wait_out(1)

        compute(1)
        fire_out(2 * g + 1, 1)
        return carry

    lax.fori_loop(0, _NPAIR, pair_body, 0)

    wait_gather(0)
    wait_out(0)
    compute(0)
    fire_out(_NCHUNK - 1, 0)
    wait_out(1)
    wait_out(0)


_pix_sc = pl.kernel(
    _sc_body,
    out_type=jax.ShapeDtypeStruct((_N_EDGES, _D), jnp.float32),
    mesh=plsc.VectorSubcoreMesh(core_axis_name="c", subcore_axis_name="s"),
    scratch_types=[
        pltpu.VMEM((_C,), jnp.int32),
        pltpu.VMEM((_C,), jnp.int32),
        pltpu.VMEM((_C,), jnp.int32),
        pltpu.VMEM((_C,), jnp.int32),
        pltpu.VMEM((_C, _D), jnp.float32),
        pltpu.VMEM((_C, _D), jnp.float32),
        pltpu.VMEM((_C, _D), jnp.float32),
        pltpu.VMEM((_C, _D), jnp.float32),
        pltpu.VMEM((_C, _D), jnp.float32),
        pltpu.VMEM((_C, _D), jnp.float32),
        pltpu.VMEM((_D,), jnp.float32),
        pltpu.VMEM((_D,), jnp.float32),
        pltpu.SemaphoreType.DMA,
        pltpu.SemaphoreType.DMA,
        pltpu.SemaphoreType.DMA,
        pltpu.SemaphoreType.DMA,
    ],
)


@jax.jit
def kernel(px, ind_2, wi, wj):
    ind_i = ind_2[:, 0]
    ind_j = ind_2[:, 1]
    return _pix_sc(px, ind_i, ind_j, wi, wj)


# speedup vs baseline: 7.6086x; 1.2573x over previous
import jax
import jax.numpy as jnp
from jax import lax
from jax.experimental import pallas as pl
from jax.experimental.pallas import tpu as pltpu
from jax.experimental.pallas import tpu_sc as plsc

_N_NODES = 10000
_N_EDGES = 320000
_D = 128
_NW = 32
_EPW = _N_EDGES // _NW
_C = 80
_NCHUNK = _EPW // _C
_NS = 3
_NTRIP = 41
_L = 16


def _sc_body(px_hbm, ind_i_hbm, ind_j_hbm, wi_hbm, wj_hbm, out_hbm,
             idx_i_all, idx_j_all,
             bi0, bj0, bi1, bj1, bi2, bj2, bo0, bo1, bo2,
             wi_v, wj_v,
             gsem0, gsem1, gsem2, osem0, osem1, osem2):
    wid = lax.axis_index("s") * 2 + lax.axis_index("c")

    pltpu.sync_copy(ind_i_hbm.at[wid], idx_i_all)
    pltpu.sync_copy(ind_j_hbm.at[wid], idx_j_all)
    pltpu.sync_copy(wi_hbm, wi_v)
    pltpu.sync_copy(wj_hbm, wj_v)
    wiv = [wi_v[pl.ds(d * _L, _L)] for d in range(_D // _L)]
    wjv = [wj_v[pl.ds(d * _L, _L)] for d in range(_D // _L)]

    base = wid * _EPW
    bufs = ((bi0, bj0), (bi1, bj1), (bi2, bj2))
    outs = (bo0, bo1, bo2)
    gsem = (gsem0, gsem1, gsem2)
    osem = (osem0, osem1, osem2)

    def fire_gather(k, s):
        pltpu.async_copy(px_hbm.at[idx_i_all.at[k]], bufs[s][0], gsem[s])
        pltpu.async_copy(px_hbm.at[idx_j_all.at[k]], bufs[s][1], gsem[s])

    def wait_gather(s):
        pltpu.make_async_copy(px_hbm.at[pl.ds(0, _C)], bufs[s][0], gsem[s]).wait()
        pltpu.make_async_copy(px_hbm.at[pl.ds(0, _C)], bufs[s][1], gsem[s]).wait()

    def compute(s):
        bi, bj = bufs[s]
        bo = outs[s]

        @plsc.parallel_loop(0, _C, unroll=4)
        def _(e):
            for d in range(_D // _L):
                sl = pl.ds(d * _L, _L)
                bo[e, sl] = wiv[d] * bi[e, sl] + wjv[d] * bj[e, sl]

    def fire_out(k, s):
        pltpu.async_copy(outs[s], out_hbm.at[pl.ds(base + k * _C, _C)], osem[s])

    def wait_out(s):
        pltpu.make_async_copy(outs[s], out_hbm.at[pl.ds(0, _C)], osem[s]).wait()

    fire_gather(0, 0)
    fire_gather(1, 1)

    def trip_body(g, carry):
        c0 = 3 * g
        for s in range(_NS):
            c = c0 + s
            fire_gather(c + 2, (s + 2) % _NS)
            wait_gather(s)

            @pl.when(g > 0)
            def _():
                wait_out(s)

            compute(s)
            fire_out(c, s)
        return carry

    lax.fori_loop(0, _NTRIP, trip_body, 0)

    for (c, s) in ((_NCHUNK - 2, 0), (_NCHUNK - 1, 1)):
        wait_gather(s)
        wait_out(s)
        compute(s)
        fire_out(c, s)
    wait_out(2)
    wait_out(0)
    wait_out(1)


_pix_sc = pl.kernel(
    _sc_body,
    out_type=jax.ShapeDtypeStruct((_N_EDGES, _D), jnp.float32),
    mesh=plsc.VectorSubcoreMesh(core_axis_name="c", subcore_axis_name="s"),
    scratch_types=(
        [pltpu.VMEM((_NCHUNK, _C), jnp.int32)] * 2
        + [pltpu.VMEM((_C, _D), jnp.float32)] * 9
        + [pltpu.VMEM((_D,), jnp.float32)] * 2
        + [pltpu.SemaphoreType.DMA] * 6
    ),
)


@jax.jit
def kernel(px, ind_2, wi, wj):
    ind_i = ind_2[:, 0].reshape(_NW, _NCHUNK, _C)
    ind_j = ind_2[:, 1].reshape(_NW, _NCHUNK, _C)
    return _pix_sc(px, ind_i, ind_j, wi, wj)
